# baseline (speedup 1.0000x reference)
import jax
import jax.numpy as jnp
import numpy as np
from jax import lax
from jax.experimental import pallas as pl
from jax.experimental.pallas import tpu as pltpu
from jax.experimental.pallas import tpu_sc as plsc

VOCAB_ = 1000
EMB_ = 64
NW = 32
B_ = 1024
QR = 20 * 20
CR = 50 * 20
NBLK = B_ // 128
NS = 6
TP = 1025
SP = 129
TV = EMB_ * TP


def _run_rows(cvecs, idx_hbm, out_hbm, tv, idxv, pbuf, stg, isem, ssem, lo,
              hi, cnt0, drain_dst):
    def row(r, cnt):
        h = pl.multiple_of((r % 2) * B_, B_)
        @pl.when(r + 1 < hi)
        def _():
            nh = ((r + 1) % 2) * B_
            pltpu.async_copy(
                idx_hbm.at[pl.ds(pl.multiple_of((r + 1) * B_, B_), B_)],
                idxv.at[pl.ds(pl.multiple_of(nh, B_), B_)],
                isem.at[(r + 1) % 2])

        pltpu.make_async_copy(
            idx_hbm.at[pl.ds(pl.multiple_of(r * B_, B_), B_)],
            idxv.at[pl.ds(h, B_)], isem.at[r % 2]).wait()

        def blk(k, cnt):
            s = cnt % NS
            b0 = pl.multiple_of(k * 128, 128)

            @pl.when(cnt >= NS)
            def _():
                pltpu.make_async_copy(stg.at[s], drain_dst, ssem.at[s]).wait()

            tvecs, evecs, svecs, zvec = cvecs

            def grp(g, gv):
                iv = idxv[pl.ds(h + k * 128 + 16 * g, 16)]
                for t in range(16):
                    bc = jnp.broadcast_to(iv[t], (16,))
                    cv = gv + t
                    for e in range(4):
                        val = plsc.load_gather(tv, [evecs[e] + bc])
                        plsc.store_scatter(pbuf, [svecs[e] + cv], val)
                return gv + 16

            lax.fori_loop(0, 8, grp, zvec)

            def erow(e, c):
                for m in range(8):
                    stg[s, e, pl.ds(16 * m, 16)] = pbuf[
                        pl.ds(e * SP + 16 * m, 16)]
                return c

            lax.fori_loop(0, EMB_, erow, 0, unroll=4)
            pltpu.async_copy(stg.at[s], out_hbm.at[r, :, pl.ds(b0, 128)],
                             ssem.at[s])
            return cnt + 1

        return lax.fori_loop(0, NBLK, blk, cnt)

    pltpu.async_copy(
        idx_hbm.at[pl.ds(pl.multiple_of(lo * B_, B_), B_)],
        idxv.at[pl.ds(pl.multiple_of((lo % 2) * B_, B_), B_)],
        isem.at[lo % 2])
    return lax.fori_loop(lo, hi, row, cnt0)


def _body(qT_hbm, cT_hbm, tT_hbm, qout_hbm, cout_hbm,
          tv, idxv, pbuf, stg, isem, ssem):
    wid = lax.axis_index("s") * 2 + lax.axis_index("c")

    pltpu.sync_copy(tT_hbm, tv)

    iota = lax.iota(jnp.int32, 16)
    zvec = iota * 0
    tvecs = [zvec + t for t in range(16)]
    evecs = [(iota + e0) * TP for e0 in range(0, 64, 16)]
    svecs = [(iota + e0) * SP for e0 in range(0, 64, 16)]
    cvecs = (tvecs, evecs, svecs, zvec)

    drain = qout_hbm.at[0, :, pl.ds(0, 128)]
    cnt = _run_rows(cvecs, qT_hbm, qout_hbm, tv, idxv, pbuf, stg, isem, ssem,
                    (wid * QR) // NW, ((wid + 1) * QR) // NW, 0, drain)
    cnt = _run_rows(cvecs, cT_hbm, cout_hbm, tv, idxv, pbuf, stg, isem, ssem,
                    (wid * CR) // NW, ((wid + 1) * CR) // NW, cnt, drain)

    for s in range(NS):
        pltpu.make_async_copy(stg.at[s], drain, ssem.at[s]).wait()


@jax.jit
def _run(qT, cT, tT):
    mesh = plsc.VectorSubcoreMesh(core_axis_name="c", subcore_axis_name="s")
    f = pl.kernel(
        _body,
        out_type=(
            jax.ShapeDtypeStruct((QR, EMB_, B_), jnp.float32),
            jax.ShapeDtypeStruct((CR, EMB_, B_), jnp.float32),
        ),
        mesh=mesh,
        scratch_types=[
            pltpu.VMEM((TV,), jnp.float32),
            pltpu.VMEM((2 * B_,), jnp.int32),
            pltpu.VMEM((EMB_ * SP,), jnp.float32),
            pltpu.VMEM((NS, EMB_, 128), jnp.float32),
            pltpu.SemaphoreType.DMA((2,)),
            pltpu.SemaphoreType.DMA((NS,)),
        ],
        compiler_params=pltpu.CompilerParams(use_tc_tiling_on_sc=True,
                                             needs_layout_passes=False),
    )
    return f(qT, cT, tT)


def kernel(question, context, char_table):
    qT = jnp.transpose(question, (1, 2, 0)).reshape(-1).astype(jnp.int32)
    cT = jnp.transpose(context, (1, 2, 0)).reshape(-1).astype(jnp.int32)
    tT = jnp.pad(char_table.T, ((0, 0), (0, TP - VOCAB_))).reshape(-1)
    qoT, coT = _run(qT, cT, tT)
    q_emb = qoT.reshape(20, 20, EMB_, B_).transpose(3, 0, 1, 2)
    c_emb = coT.reshape(50, 20, EMB_, B_).transpose(3, 0, 1, 2)
    return (q_emb, c_emb)


# speedup vs baseline: 1.6662x; 1.6662x over previous
import jax
import jax.numpy as jnp
from jax import lax
from jax.experimental import pallas as pl
from jax.experimental.pallas import tpu as pltpu
from jax.experimental.pallas import tpu_sc as plsc

VOCAB_ = 1000
EMB_ = 64
NW = 32
B_ = 1024
QR = 20 * 20
CR = 50 * 20
NBLK = B_ // 128
NS = 7
TV = EMB_ * B_


def _run_rows(idx_hbm, out_hbm, tv, idxv, stg, isem, ssem, lo, hi, cnt0,
              drain_dst):
    def row(r, cnt):
        h = pl.multiple_of((r % 2) * B_, B_)
        @pl.when(r + 1 < hi)
        def _():
            nh = ((r + 1) % 2) * B_
            pltpu.async_copy(
                idx_hbm.at[pl.ds(pl.multiple_of((r + 1) * B_, B_), B_)],
                idxv.at[pl.ds(pl.multiple_of(nh, B_), B_)],
                isem.at[(r + 1) % 2])

        pltpu.make_async_copy(
            idx_hbm.at[pl.ds(pl.multiple_of(r * B_, B_), B_)],
            idxv.at[pl.ds(h, B_)], isem.at[r % 2]).wait()

        def blk(k, cnt):
            s = cnt % NS
            b0 = pl.multiple_of(k * 128, 128)
            iv = [idxv[pl.ds(h + k * 128 + 16 * j, 16)] for j in range(8)]

            @pl.when(cnt >= NS)
            def _():
                pltpu.make_async_copy(stg.at[s], drain_dst, ssem.at[s]).wait()

            def erow(e, c):
                rowref = tv.at[pl.ds(pl.multiple_of(e * B_, B_), B_)]
                for j in range(8):
                    stg[s, e, pl.ds(16 * j, 16)] = plsc.load_gather(
                        rowref, [iv[j]])
                return c

            lax.fori_loop(0, EMB_, erow, 0, unroll=8)
            pltpu.async_copy(stg.at[s], out_hbm.at[r, :, pl.ds(b0, 128)],
                             ssem.at[s])
            return cnt + 1

        return lax.fori_loop(0, NBLK, blk, cnt)

    pltpu.async_copy(
        idx_hbm.at[pl.ds(pl.multiple_of(lo * B_, B_), B_)],
        idxv.at[pl.ds(pl.multiple_of((lo % 2) * B_, B_), B_)],
        isem.at[lo % 2])
    return lax.fori_loop(lo, hi, row, cnt0)


def _body(qT_hbm, cT_hbm, tT_hbm, qout_hbm, cout_hbm,
          tv, idxv, stg, isem, ssem):
    wid = lax.axis_index("s") * 2 + lax.axis_index("c")

    pltpu.sync_copy(tT_hbm, tv)

    drain = qout_hbm.at[0, :, pl.ds(0, 128)]
    cnt = _run_rows(qT_hbm, qout_hbm, tv, idxv, stg, isem, ssem,
                    (wid * QR) // NW, ((wid + 1) * QR) // NW, 0, drain)
    cnt = _run_rows(cT_hbm, cout_hbm, tv, idxv, stg, isem, ssem,
                    (wid * CR) // NW, ((wid + 1) * CR) // NW, cnt, drain)

    for s in range(NS):
        pltpu.make_async_copy(stg.at[s], drain, ssem.at[s]).wait()


@jax.jit
def _run(qT, cT, tT):
    mesh = plsc.VectorSubcoreMesh(core_axis_name="c", subcore_axis_name="s")
    f = pl.kernel(
        _body,
        out_type=(
            jax.ShapeDtypeStruct((QR, EMB_, B_), jnp.float32),
            jax.ShapeDtypeStruct((CR, EMB_, B_), jnp.float32),
        ),
        mesh=mesh,
        scratch_types=[
            pltpu.VMEM((TV,), jnp.float32),
            pltpu.VMEM((2 * B_,), jnp.int32),
            pltpu.VMEM((NS, EMB_, 128), jnp.float32),
            pltpu.SemaphoreType.DMA((2,)),
            pltpu.SemaphoreType.DMA((NS,)),
        ],
        compiler_params=pltpu.CompilerParams(use_tc_tiling_on_sc=True,
                                             needs_layout_passes=False),
    )
    return f(qT, cT, tT)


def kernel(question, context, char_table):
    qT = jnp.transpose(question, (1, 2, 0)).reshape(-1).astype(jnp.int32)
    cT = jnp.transpose(context, (1, 2, 0)).reshape(-1).astype(jnp.int32)
    tT = jnp.pad(char_table.T, ((0, 0), (0, B_ - VOCAB_))).reshape(-1)
    qoT, coT = _run(qT, cT, tT)
    q_emb = qoT.reshape(20, 20, EMB_, B_).transpose(3, 0, 1, 2)
    c_emb = coT.reshape(50, 20, EMB_, B_).transpose(3, 0, 1, 2)
    return (q_emb, c_emb)


# speedup vs baseline: 1.7058x; 1.0238x over previous
import jax
import jax.numpy as jnp
from jax import lax
from jax.experimental import pallas as pl
from jax.experimental.pallas import tpu as pltpu
from jax.experimental.pallas import tpu_sc as plsc

VOCAB_ = 1000
EMB_ = 64
NW = 32
B_ = 1024
QR = 20 * 20
CR = 50 * 20
NBLK = B_ // 128
NS = 7
TV = EMB_ * B_


def _run_rows(idx_hbm, out_hbm, tv, idxv, stg, isem, ssem, lo, hi, cnt0,
              drain_dst):
    def row(r, cnt):
        h = pl.multiple_of((r % 2) * B_, B_)
        @pl.when(r + 1 < hi)
        def _():
            nh = ((r + 1) % 2) * B_
            pltpu.async_copy(
                idx_hbm.at[pl.ds(pl.multiple_of((r + 1) * B_, B_), B_)],
                idxv.at[pl.ds(pl.multiple_of(nh, B_), B_)],
                isem.at[(r + 1) % 2])

        pltpu.make_async_copy(
            idx_hbm.at[pl.ds(pl.multiple_of(r * B_, B_), B_)],
            idxv.at[pl.ds(h, B_)], isem.at[r % 2]).wait()

        def blk(k, cnt):
            s = cnt % NS
            b0 = pl.multiple_of(k * 128, 128)
            iv = [idxv[pl.ds(h + k * 128 + 16 * j, 16)] for j in range(8)]

            @pl.when(cnt >= NS)
            def _():
                pltpu.make_async_copy(stg.at[s], drain_dst, ssem.at[s]).wait()

            def erow(e, c):
                rowref = tv.at[pl.ds(pl.multiple_of(e * B_, B_), B_)]
                for j in range(8):
                    stg[s, e, pl.ds(16 * j, 16)] = plsc.load_gather(
                        rowref, [iv[j]])
                return c

            lax.fori_loop(0, EMB_, erow, 0, unroll=8)
            pltpu.async_copy(stg.at[s], out_hbm.at[r, :, pl.ds(b0, 128)],
                             ssem.at[s])
            return cnt + 1

        return lax.fori_loop(0, NBLK, blk, cnt)

    @pl.when(lo < hi)
    def _():
        pltpu.async_copy(
            idx_hbm.at[pl.ds(pl.multiple_of(lo * B_, B_), B_)],
            idxv.at[pl.ds(pl.multiple_of((lo % 2) * B_, B_), B_)],
            isem.at[lo % 2])
    return lax.fori_loop(lo, hi, row, cnt0)


def _body(qT_hbm, cT_hbm, tT_hbm, qout_hbm, cout_hbm,
          tv, idxv, stg, isem, ssem):
    wid = lax.axis_index("s") * 2 + lax.axis_index("c")

    pltpu.sync_copy(tT_hbm, tv)

    lo = (wid * (QR + CR)) // NW
    hi = ((wid + 1) * (QR + CR)) // NW
    drain = qout_hbm.at[0, :, pl.ds(0, 128)]
    cnt = _run_rows(qT_hbm, qout_hbm, tv, idxv, stg, isem, ssem,
                    jnp.minimum(lo, QR), jnp.minimum(hi, QR), 0, drain)
    cnt = _run_rows(cT_hbm, cout_hbm, tv, idxv, stg, isem, ssem,
                    jnp.maximum(lo, QR) - QR, jnp.maximum(hi, QR) - QR,
                    cnt, drain)

    for s in range(NS):
        pltpu.make_async_copy(stg.at[s], drain, ssem.at[s]).wait()


@jax.jit
def _run(qT, cT, tT):
    mesh = plsc.VectorSubcoreMesh(core_axis_name="c", subcore_axis_name="s")
    f = pl.kernel(
        _body,
        out_type=(
            jax.ShapeDtypeStruct((QR, EMB_, B_), jnp.float32),
            jax.ShapeDtypeStruct((CR, EMB_, B_), jnp.float32),
        ),
        mesh=mesh,
        scratch_types=[
            pltpu.VMEM((TV,), jnp.float32),
            pltpu.VMEM((2 * B_,), jnp.int32),
            pltpu.VMEM((NS, EMB_, 128), jnp.float32),
            pltpu.SemaphoreType.DMA((2,)),
            pltpu.SemaphoreType.DMA((NS,)),
        ],
        compiler_params=pltpu.CompilerParams(use_tc_tiling_on_sc=True,
                                             needs_layout_passes=False),
    )
    return f(qT, cT, tT)


def kernel(question, context, char_table):
    qT = jnp.transpose(question, (1, 2, 0)).reshape(-1).astype(jnp.int32)
    cT = jnp.transpose(context, (1, 2, 0)).reshape(-1).astype(jnp.int32)
    tT = jnp.pad(char_table.T, ((0, 0), (0, B_ - VOCAB_))).reshape(-1)
    qoT, coT = _run(qT, cT, tT)
    q_emb = qoT.reshape(20, 20, EMB_, B_).transpose(3, 0, 1, 2)
    c_emb = coT.reshape(50, 20, EMB_, B_).transpose(3, 0, 1, 2)
    return (q_emb, c_emb)
